# baseline (speedup 1.0000x reference)
import functools
import jax
import jax.numpy as jnp
from jax.experimental import pallas as pl
from jax.experimental.pallas import tpu as pltpu

NUM_NODES = 10000
NUM_REL = 8

_EB = 2000


def _edge_mlp_body(u_ref, t_ref, w_ref, mb1_ref, w2_ref, mb2_ref, o_ref):
    u = u_ref[0]
    t = t_ref[0]
    hid = jnp.maximum(u + jnp.dot(t, w_ref[...],
                                  preferred_element_type=jnp.float32)
                      + mb1_ref[...], 0.0)
    lg = jnp.dot(hid, w2_ref[...], preferred_element_type=jnp.float32)
    o_ref[0, 0, :] = lg[:, 0] + mb2_ref[0]


def _edge_mlp(u, etext, w_t, mb1, mW2, mb2):
    E = u.shape[0]
    nb = E // _EB
    u3 = u.reshape(nb, _EB, 128)
    t3 = etext.reshape(nb, _EB, 128)
    out = pl.pallas_call(
        _edge_mlp_body,
        grid=(nb,),
        in_specs=[
            pl.BlockSpec((1, _EB, 128), lambda i: (i, 0, 0)),
            pl.BlockSpec((1, _EB, 128), lambda i: (i, 0, 0)),
            pl.BlockSpec((128, 128), lambda i: (0, 0)),
            pl.BlockSpec((128,), lambda i: (0,)),
            pl.BlockSpec((128, 1), lambda i: (0, 0)),
            pl.BlockSpec((1,), lambda i: (0,)),
        ],
        out_specs=pl.BlockSpec((1, 1, _EB), lambda i: (i, 0, 0)),
        out_shape=jax.ShapeDtypeStruct((nb, 1, _EB), jnp.float32),
    )(u3, t3, w_t, mb1, mW2, mb2)
    return out.reshape(E)


def _rgcn_layer(x, src, dst, edge_type, Wr, Wroot, b):
    N = NUM_NODES
    key = dst * NUM_REL + edge_type
    msg = x[src]
    seg = jax.ops.segment_sum(msg, key, num_segments=N * NUM_REL)
    cnt = jax.ops.segment_sum(jnp.ones((msg.shape[0], 1), msg.dtype), key,
                              num_segments=N * NUM_REL)
    mean = (seg / jnp.maximum(cnt, 1.0)).reshape(N, NUM_REL, x.shape[1])
    return jnp.einsum('nri,rio->no', mean, Wr) + x @ Wroot + b


def kernel(edge_index, edge_type, e_text_emb, node_emb, Wr1, Wroot1, b1,
           Wr2, Wroot2, b2, mW1, mb1, mW2, mb2):
    src, dst = edge_index[0], edge_index[1]
    x0 = node_emb
    h = jax.nn.relu(_rgcn_layer(x0, src, dst, edge_type, Wr1, Wroot1, b1))
    h = jax.nn.relu(_rgcn_layer(h, src, dst, edge_type, Wr2, Wroot2, b2))
    P = h @ mW1[:128]
    Q = h @ mW1[128:256]
    u = P[src] + Q[dst]
    logits = _edge_mlp(u, e_text_emb, mW1[256:], mb1, mW2, mb2)
    return (logits, h)


# speedup vs baseline: 4.2115x; 4.2115x over previous
import functools
import jax
import jax.numpy as jnp
from jax import lax
from jax.experimental import pallas as pl
from jax.experimental.pallas import tpu as pltpu
from jax.experimental.pallas import tpu_sc as plsc

N = 10000
R = 8
E = 320000
NW = 32
EPW = E // NW
NB = 125
BE = 80
HB = 81920
NP = 10240

_mesh = plsc.VectorSubcoreMesh(core_axis_name="c", subcore_axis_name="s")


def _wid():
    return lax.axis_index("c") * 16 + lax.axis_index("s")


@functools.partial(
    pl.kernel,
    out_type=jax.ShapeDtypeStruct((2, HB), jnp.float32),
    mesh=_mesh,
    scratch_types=[
        pltpu.VMEM((NB, BE), jnp.int32),
        pltpu.VMEM((BE,), jnp.float32),
        pltpu.VMEM((5120,), jnp.float32),
        pltpu.VMEM_SHARED((HB,), jnp.float32),
    ],
)
def _hist_k(key_hbm, out_hbm, kb, ones, zb, hist):
    c = lax.axis_index("c")
    s = lax.axis_index("s")
    w = _wid()

    def fill_ones(i, _):
        ones[pl.ds(i * 16, 16)] = jnp.ones((16,), jnp.float32)
        return 0

    lax.fori_loop(0, BE // 16, fill_ones, 0)

    def fill_z(i, _):
        zb[pl.ds(i * 16, 16)] = jnp.zeros((16,), jnp.float32)
        return 0

    lax.fori_loop(0, 5120 // 16, fill_z, 0)
    pltpu.sync_copy(zb, hist.at[pl.ds(s * 5120, 5120)])
    plsc.subcore_barrier()

    pltpu.sync_copy(key_hbm.at[w], kb)

    def body(j, _):
        pltpu.sync_copy(ones, hist.at[kb.at[j]], add=True)
        return 0

    lax.fori_loop(0, NB, body, 0)
    plsc.subcore_barrier()
    pltpu.sync_copy(hist.at[pl.ds(s * 5120, 5120)],
                    out_hbm.at[c, pl.ds(s * 5120, 5120)])


@functools.partial(
    pl.kernel,
    out_type=jax.ShapeDtypeStruct((NW, NB, BE), jnp.float32),
    mesh=_mesh,
    scratch_types=[
        pltpu.VMEM((NB, BE), jnp.int32),
        pltpu.VMEM((NB, BE), jnp.float32),
        pltpu.SemaphoreType.DMA,
    ],
)
def _scale_k(key_hbm, invc_hbm, out_hbm, kb, sb, sem):
    w = _wid()
    pltpu.sync_copy(key_hbm.at[w], kb)

    def body(j, _):
        pltpu.async_copy(invc_hbm.at[kb.at[j]], sb.at[j], sem).wait()
        return 0

    lax.fori_loop(0, NB, body, 0)
    pltpu.sync_copy(sb, out_hbm.at[w])


@functools.partial(
    pl.kernel,
    out_type=jax.ShapeDtypeStruct((2, NP, 128), jnp.float32),
    mesh=_mesh,
    scratch_types=[
        pltpu.VMEM((25, BE), jnp.int32),
        pltpu.VMEM((25, BE), jnp.int32),
        pltpu.VMEM((25, BE), jnp.float32),
        pltpu.VMEM((BE, 128), jnp.float32),
        pltpu.VMEM((16, 128), jnp.float32),
        pltpu.VMEM_SHARED((NP, 128), jnp.float32),
        pltpu.SemaphoreType.DMA,
    ],
)
def _agg_k(y_hbm, iy_hbm, dst_hbm, sc_hbm, out_hbm,
           iyb, db, sb, rb, zb, acc, sem):
    c = lax.axis_index("c")
    s = lax.axis_index("s")
    w = _wid()

    def fill_z(i, _):
        row = i // 8
        f = i % 8
        zb[row, pl.ds(f * 16, 16)] = jnp.zeros((16,), jnp.float32)
        return 0

    lax.fori_loop(0, 16 * 8, fill_z, 0)

    def zcopy(i, _):
        pltpu.sync_copy(zb, acc.at[pl.ds(s * 640 + i * 16, 16)])
        return 0

    lax.fori_loop(0, 40, zcopy, 0)
    plsc.subcore_barrier()

    def chunk(cc, _):
        pltpu.sync_copy(iy_hbm.at[w, cc], iyb)
        pltpu.sync_copy(dst_hbm.at[w, cc], db)
        pltpu.sync_copy(sc_hbm.at[w, cc], sb)

        def body(j, _):
            pltpu.async_copy(y_hbm.at[iyb.at[j]], rb, sem).wait()

            def escale(i, _):
                sc16 = sb[j, pl.ds(i * 16, 16)]
                for ee in range(16):
                    e = i * 16 + ee
                    sc = sc16[ee]
                    for f in range(8):
                        rb[e, pl.ds(f * 16, 16)] = (
                            rb[e, pl.ds(f * 16, 16)] * sc)
                return 0

            lax.fori_loop(0, BE // 16, escale, 0)
            pltpu.sync_copy(rb, acc.at[db.at[j]], add=True)
            return 0

        lax.fori_loop(0, 25, body, 0)
        return 0

    lax.fori_loop(0, 5, chunk, 0)
    plsc.subcore_barrier()
    pltpu.sync_copy(acc.at[pl.ds(s * 640, 640)],
                    out_hbm.at[c, pl.ds(s * 640, 640)])


@functools.partial(
    pl.kernel,
    out_type=(jax.ShapeDtypeStruct((E, 128), jnp.float32),
              jax.ShapeDtypeStruct((E, 128), jnp.float32)),
    mesh=_mesh,
    scratch_types=[
        pltpu.VMEM((NB, BE), jnp.int32),
        pltpu.VMEM((NB, BE), jnp.int32),
        pltpu.VMEM((BE, 128), jnp.float32),
        pltpu.VMEM((BE, 128), jnp.float32),
        pltpu.SemaphoreType.DMA,
        pltpu.SemaphoreType.DMA,
    ],
)
def _gath_k(pq_hbm, ip_hbm, iq_hbm, up_hbm, uq_hbm,
            ipb, iqb, pb, qb, sp, sq):
    w = _wid()
    pltpu.sync_copy(ip_hbm.at[w], ipb)
    pltpu.sync_copy(iq_hbm.at[w], iqb)

    def body(j, _):
        cp = pltpu.async_copy(pq_hbm.at[ipb.at[j]], pb, sp)
        cq = pltpu.async_copy(pq_hbm.at[iqb.at[j]], qb, sq)
        cp.wait()
        cq.wait()
        base = w * EPW + j * BE
        pltpu.sync_copy(pb, up_hbm.at[pl.ds(base, BE)])
        pltpu.sync_copy(qb, uq_hbm.at[pl.ds(base, BE)])
        return 0

    lax.fori_loop(0, NB, body, 0)


def _inv_body(p_ref, o_ref):
    o_ref[...] = 1.0 / jnp.maximum(p_ref[0] + p_ref[1], 1.0)


def _inv_k(hpart):
    h3 = hpart.reshape(2, 640, 128)
    out = pl.pallas_call(
        _inv_body,
        grid=(5,),
        in_specs=[pl.BlockSpec((2, 128, 128), lambda i: (0, i, 0))],
        out_specs=pl.BlockSpec((128, 128), lambda i: (i, 0)),
        out_shape=jax.ShapeDtypeStruct((640, 128), jnp.float32),
    )(h3)
    return out.reshape(HB)


def _ymul_body(x_ref, w_ref, o_ref):
    o_ref[0] = jnp.dot(x_ref[...], w_ref[0],
                       preferred_element_type=jnp.float32)


def _ymul_k(x, Wr):
    out = pl.pallas_call(
        _ymul_body,
        grid=(R, 5),
        in_specs=[
            pl.BlockSpec((2000, 128), lambda r, i: (i, 0)),
            pl.BlockSpec((1, 128, 128), lambda r, i: (r, 0, 0)),
        ],
        out_specs=pl.BlockSpec((1, 2000, 128), lambda r, i: (r, i, 0)),
        out_shape=jax.ShapeDtypeStruct((R, N, 128), jnp.float32),
    )(x, Wr)
    return out.reshape(R * N, 128)


def _dense_body(a_ref, x_ref, wr_ref, b_ref, o_ref):
    o_ref[...] = jnp.maximum(
        a_ref[0] + a_ref[1]
        + jnp.dot(x_ref[...], wr_ref[...], preferred_element_type=jnp.float32)
        + b_ref[...], 0.0)


def _dense_k(apart, x, Wroot, b):
    return pl.pallas_call(
        _dense_body,
        grid=(5,),
        in_specs=[
            pl.BlockSpec((2, 2000, 128), lambda i: (0, i, 0)),
            pl.BlockSpec((2000, 128), lambda i: (i, 0)),
            pl.BlockSpec((128, 128), lambda i: (0, 0)),
            pl.BlockSpec((128,), lambda i: (0,)),
        ],
        out_specs=pl.BlockSpec((2000, 128), lambda i: (i, 0)),
        out_shape=jax.ShapeDtypeStruct((N, 128), jnp.float32),
    )(apart, x, Wroot, b)


def _pq_body(x_ref, w_ref, o_ref):
    o_ref[...] = jnp.dot(x_ref[...], w_ref[...],
                         preferred_element_type=jnp.float32)


def _pq_k(h, Wpq):
    return pl.pallas_call(
        _pq_body,
        grid=(5,),
        in_specs=[
            pl.BlockSpec((2000, 128), lambda i: (i, 0)),
            pl.BlockSpec((128, 256), lambda i: (0, 0)),
        ],
        out_specs=pl.BlockSpec((2000, 256), lambda i: (i, 0)),
        out_shape=jax.ShapeDtypeStruct((N, 256), jnp.float32),
    )(h, Wpq)


_EB = 2000


def _mlp_body(up_ref, uq_ref, t_ref, w_ref, mb1_ref, w2_ref, mb2_ref, o_ref):
    u = up_ref[0] + uq_ref[0]
    t = t_ref[0]
    hid = jnp.maximum(u + jnp.dot(t, w_ref[...],
                                  preferred_element_type=jnp.float32)
                      + mb1_ref[...], 0.0)
    lg = jnp.dot(hid, w2_ref[...], preferred_element_type=jnp.float32)
    o_ref[0, 0, :] = lg[:, 0] + mb2_ref[0]


def _mlp_k(up, uq, etext, w_t, mb1, mW2, mb2):
    nb = E // _EB
    u3p = up.reshape(nb, _EB, 128)
    u3q = uq.reshape(nb, _EB, 128)
    t3 = etext.reshape(nb, _EB, 128)
    out = pl.pallas_call(
        _mlp_body,
        grid=(nb,),
        in_specs=[
            pl.BlockSpec((1, _EB, 128), lambda i: (i, 0, 0)),
            pl.BlockSpec((1, _EB, 128), lambda i: (i, 0, 0)),
            pl.BlockSpec((1, _EB, 128), lambda i: (i, 0, 0)),
            pl.BlockSpec((128, 128), lambda i: (0, 0)),
            pl.BlockSpec((128,), lambda i: (0,)),
            pl.BlockSpec((128, 1), lambda i: (0, 0)),
            pl.BlockSpec((1,), lambda i: (0,)),
        ],
        out_specs=pl.BlockSpec((1, 1, _EB), lambda i: (i, 0, 0)),
        out_shape=jax.ShapeDtypeStruct((nb, 1, _EB), jnp.float32),
    )(u3p, u3q, t3, w_t, mb1, mW2, mb2)
    return out.reshape(E)


def kernel(edge_index, edge_type, e_text_emb, node_emb, Wr1, Wroot1, b1,
           Wr2, Wroot2, b2, mW1, mb1, mW2, mb2):
    src = edge_index[0]
    dst = edge_index[1]
    et = edge_type

    key3 = (dst * R + et).astype(jnp.int32).reshape(NW, NB, BE)
    iy3 = (et * N + src).astype(jnp.int32).reshape(NW, NB, BE)
    dst3 = dst.astype(jnp.int32).reshape(NW, NB, BE)
    ip3 = (2 * src).astype(jnp.int32).reshape(NW, NB, BE)
    iq3 = (2 * dst + 1).astype(jnp.int32).reshape(NW, NB, BE)

    hpart = _hist_k(key3)
    invc = _inv_k(hpart)
    scale3 = _scale_k(key3, invc)
    iy4 = iy3.reshape(NW, 5, 25, BE)
    dst4 = dst3.reshape(NW, 5, 25, BE)
    scale4 = scale3.reshape(NW, 5, 25, BE)

    x0 = node_emb
    Y1 = _ymul_k(x0, Wr1)
    a1 = _agg_k(Y1, iy4, dst4, scale4)
    h1 = _dense_k(a1, x0, Wroot1, b1)

    Y2 = _ymul_k(h1, Wr2)
    a2 = _agg_k(Y2, iy4, dst4, scale4)
    h = _dense_k(a2, h1, Wroot2, b2)

    Wpq = jnp.concatenate([mW1[:128], mW1[128:256]], axis=1)
    PQ2 = _pq_k(h, Wpq).reshape(2 * N, 128)
    up, uq = _gath_k(PQ2, ip3, iq3)
    logits = _mlp_k(up, uq, e_text_emb, mW1[256:], mb1, mW2, mb2)
    return (logits, h)


# speedup vs baseline: 4.6741x; 1.1098x over previous
import functools
import jax
import jax.numpy as jnp
from jax import lax
from jax.experimental import pallas as pl
from jax.experimental.pallas import tpu as pltpu
from jax.experimental.pallas import tpu_sc as plsc

N = 10000
R = 8
E = 320000
NW = 32
EPW = E // NW
NB = 125
BE = 80
HB = 81920
NP = 10240

_mesh = plsc.VectorSubcoreMesh(core_axis_name="c", subcore_axis_name="s")


def _wid():
    return lax.axis_index("c") * 16 + lax.axis_index("s")


@functools.partial(
    pl.kernel,
    out_type=jax.ShapeDtypeStruct((2, HB), jnp.float32),
    mesh=_mesh,
    scratch_types=[
        pltpu.VMEM((NB, BE), jnp.int32),
        pltpu.VMEM((BE,), jnp.float32),
        pltpu.VMEM((5120,), jnp.float32),
        pltpu.VMEM_SHARED((HB,), jnp.float32),
    ],
)
def _hist_k(key_hbm, out_hbm, kb, ones, zb, hist):
    c = lax.axis_index("c")
    s = lax.axis_index("s")
    w = _wid()

    def fill_ones(i, _):
        ones[pl.ds(i * 16, 16)] = jnp.ones((16,), jnp.float32)
        return 0

    lax.fori_loop(0, BE // 16, fill_ones, 0)

    def fill_z(i, _):
        zb[pl.ds(i * 16, 16)] = jnp.zeros((16,), jnp.float32)
        return 0

    lax.fori_loop(0, 5120 // 16, fill_z, 0)
    pltpu.sync_copy(zb, hist.at[pl.ds(s * 5120, 5120)])
    plsc.subcore_barrier()

    pltpu.sync_copy(key_hbm.at[w], kb)

    def body(j, _):
        pltpu.sync_copy(ones, hist.at[kb.at[j]], add=True)
        return 0

    lax.fori_loop(0, NB, body, 0)
    plsc.subcore_barrier()
    pltpu.sync_copy(hist.at[pl.ds(s * 5120, 5120)],
                    out_hbm.at[c, pl.ds(s * 5120, 5120)])


@functools.partial(
    pl.kernel,
    out_type=jax.ShapeDtypeStruct((NW, NB, BE), jnp.float32),
    mesh=_mesh,
    scratch_types=[
        pltpu.VMEM((NB, BE), jnp.int32),
        pltpu.VMEM((NB, BE), jnp.float32),
        pltpu.SemaphoreType.DMA,
    ],
)
def _scale_k(key_hbm, invc_hbm, out_hbm, kb, sb, sem):
    w = _wid()
    pltpu.sync_copy(key_hbm.at[w], kb)

    def body(j, _):
        pltpu.async_copy(invc_hbm.at[kb.at[j]], sb.at[j], sem).wait()
        return 0

    lax.fori_loop(0, NB, body, 0)
    pltpu.sync_copy(sb, out_hbm.at[w])


@functools.partial(
    pl.kernel,
    out_type=jax.ShapeDtypeStruct((2, NP, 128), jnp.float32),
    mesh=_mesh,
    scratch_types=[
        pltpu.VMEM((25, BE), jnp.int32),
        pltpu.VMEM((25, BE), jnp.int32),
        pltpu.VMEM((25, BE), jnp.float32),
        pltpu.VMEM((BE, 128), jnp.float32),
        pltpu.VMEM((16, 128), jnp.float32),
        pltpu.VMEM_SHARED((NP, 128), jnp.float32),
        pltpu.SemaphoreType.DMA,
    ],
)
def _agg_k(y_hbm, iy_hbm, dst_hbm, sc_hbm, out_hbm,
           iyb, db, sb, rb, zb, acc, sem):
    c = lax.axis_index("c")
    s = lax.axis_index("s")
    w = _wid()

    def fill_z(i, _):
        row = i // 8
        f = i % 8
        zb[row, pl.ds(f * 16, 16)] = jnp.zeros((16,), jnp.float32)
        return 0

    lax.fori_loop(0, 16 * 8, fill_z, 0)

    def zcopy(i, _):
        pltpu.sync_copy(zb, acc.at[pl.ds(s * 640 + i * 16, 16)])
        return 0

    lax.fori_loop(0, 40, zcopy, 0)
    plsc.subcore_barrier()

    def chunk(cc, _):
        pltpu.sync_copy(iy_hbm.at[w, cc], iyb)
        pltpu.sync_copy(dst_hbm.at[w, cc], db)
        pltpu.sync_copy(sc_hbm.at[w, cc], sb)

        def body(j, _):
            pltpu.async_copy(y_hbm.at[iyb.at[j]], rb, sem).wait()

            def escale(i, _):
                sc16 = sb[j, pl.ds(i * 16, 16)]
                for ee in range(16):
                    e = i * 16 + ee
                    sc = sc16[ee]
                    for f in range(8):
                        rb[e, pl.ds(f * 16, 16)] = (
                            rb[e, pl.ds(f * 16, 16)] * sc)
                return 0

            lax.fori_loop(0, BE // 16, escale, 0)
            pltpu.sync_copy(rb, acc.at[db.at[j]], add=True)
            return 0

        lax.fori_loop(0, 25, body, 0)
        return 0

    lax.fori_loop(0, 5, chunk, 0)
    plsc.subcore_barrier()
    pltpu.sync_copy(acc.at[pl.ds(s * 640, 640)],
                    out_hbm.at[c, pl.ds(s * 640, 640)])


@functools.partial(
    pl.kernel,
    out_type=jax.ShapeDtypeStruct((E, 128), jnp.float32),
    mesh=_mesh,
    scratch_types=[
        pltpu.VMEM((25, BE), jnp.int32),
        pltpu.VMEM((25, BE), jnp.int32),
        pltpu.VMEM((5, BE, 128), jnp.float32),
        pltpu.VMEM((5, BE, 128), jnp.float32),
        pltpu.SemaphoreType.DMA,
        pltpu.SemaphoreType.DMA,
        pltpu.SemaphoreType.DMA,
    ],
)
def _gath_k(pq_hbm, ip_hbm, iq_hbm, u_hbm,
            ipb, iqb, pb, qb, sp, sq, sw):
    w = _wid()

    def chunk(cc, _):
        pltpu.sync_copy(ip_hbm.at[w, cc], ipb)
        pltpu.sync_copy(iq_hbm.at[w, cc], iqb)

        def sub(dd, _):
            j0 = dd * 5
            cps = []
            cqs = []
            for b in range(5):
                cps.append(pltpu.async_copy(
                    pq_hbm.at[ipb.at[j0 + b]], pb.at[b], sp))
                cqs.append(pltpu.async_copy(
                    pq_hbm.at[iqb.at[j0 + b]], qb.at[b], sq))
            cws = []
            for b in range(5):
                cps[b].wait()
                cqs[b].wait()

                def eadd(e, _):
                    for f in range(8):
                        pb[b, e, pl.ds(f * 16, 16)] = (
                            pb[b, e, pl.ds(f * 16, 16)]
                            + qb[b, e, pl.ds(f * 16, 16)])
                    return 0

                lax.fori_loop(0, BE, eadd, 0)
                base = w * EPW + (cc * 25 + j0 + b) * BE
                cws.append(pltpu.async_copy(
                    pb.at[b], u_hbm.at[pl.ds(base, BE)], sw))
            for b in range(5):
                cws[b].wait()
            return 0

        lax.fori_loop(0, 5, sub, 0)
        return 0

    lax.fori_loop(0, 5, chunk, 0)


def _inv_body(p_ref, o_ref):
    o_ref[...] = 1.0 / jnp.maximum(p_ref[0] + p_ref[1], 1.0)


def _inv_k(hpart):
    h3 = hpart.reshape(2, 640, 128)
    out = pl.pallas_call(
        _inv_body,
        grid=(5,),
        in_specs=[pl.BlockSpec((2, 128, 128), lambda i: (0, i, 0))],
        out_specs=pl.BlockSpec((128, 128), lambda i: (i, 0)),
        out_shape=jax.ShapeDtypeStruct((640, 128), jnp.float32),
    )(h3)
    return out.reshape(HB)


def _ymul_body(x_ref, w_ref, o_ref):
    o_ref[0] = jnp.dot(x_ref[...], w_ref[0],
                       preferred_element_type=jnp.float32)


def _ymul_k(x, Wr):
    out = pl.pallas_call(
        _ymul_body,
        grid=(R, 5),
        in_specs=[
            pl.BlockSpec((2000, 128), lambda r, i: (i, 0)),
            pl.BlockSpec((1, 128, 128), lambda r, i: (r, 0, 0)),
        ],
        out_specs=pl.BlockSpec((1, 2000, 128), lambda r, i: (r, i, 0)),
        out_shape=jax.ShapeDtypeStruct((R, N, 128), jnp.float32),
    )(x, Wr)
    return out.reshape(R * N, 128)


def _dense_body(a_ref, x_ref, wr_ref, b_ref, o_ref):
    o_ref[...] = jnp.maximum(
        a_ref[0] + a_ref[1]
        + jnp.dot(x_ref[...], wr_ref[...], preferred_element_type=jnp.float32)
        + b_ref[...], 0.0)


def _dense_k(apart, x, Wroot, b):
    return pl.pallas_call(
        _dense_body,
        grid=(5,),
        in_specs=[
            pl.BlockSpec((2, 2000, 128), lambda i: (0, i, 0)),
            pl.BlockSpec((2000, 128), lambda i: (i, 0)),
            pl.BlockSpec((128, 128), lambda i: (0, 0)),
            pl.BlockSpec((128,), lambda i: (0,)),
        ],
        out_specs=pl.BlockSpec((2000, 128), lambda i: (i, 0)),
        out_shape=jax.ShapeDtypeStruct((N, 128), jnp.float32),
    )(apart, x, Wroot, b)


def _pq_body(x_ref, w_ref, o_ref):
    o_ref[...] = jnp.dot(x_ref[...], w_ref[...],
                         preferred_element_type=jnp.float32)


def _pq_k(h, Wpq):
    return pl.pallas_call(
        _pq_body,
        grid=(5,),
        in_specs=[
            pl.BlockSpec((2000, 128), lambda i: (i, 0)),
            pl.BlockSpec((128, 256), lambda i: (0, 0)),
        ],
        out_specs=pl.BlockSpec((2000, 256), lambda i: (i, 0)),
        out_shape=jax.ShapeDtypeStruct((N, 256), jnp.float32),
    )(h, Wpq)


_EB = 2000


def _mlp_body(u_ref, t_ref, w_ref, mb1_ref, w2_ref, mb2_ref, o_ref):
    u = u_ref[0]
    t = t_ref[0]
    hid = jnp.maximum(u + jnp.dot(t, w_ref[...],
                                  preferred_element_type=jnp.float32)
                      + mb1_ref[...], 0.0)
    lg = jnp.dot(hid, w2_ref[...], preferred_element_type=jnp.float32)
    o_ref[0, 0, :] = lg[:, 0] + mb2_ref[0]


def _mlp_k(u, etext, w_t, mb1, mW2, mb2):
    nb = E // _EB
    u3 = u.reshape(nb, _EB, 128)
    t3 = etext.reshape(nb, _EB, 128)
    out = pl.pallas_call(
        _mlp_body,
        grid=(nb,),
        in_specs=[
            pl.BlockSpec((1, _EB, 128), lambda i: (i, 0, 0)),
            pl.BlockSpec((1, _EB, 128), lambda i: (i, 0, 0)),
            pl.BlockSpec((128, 128), lambda i: (0, 0)),
            pl.BlockSpec((128,), lambda i: (0,)),
            pl.BlockSpec((128, 1), lambda i: (0, 0)),
            pl.BlockSpec((1,), lambda i: (0,)),
        ],
        out_specs=pl.BlockSpec((1, 1, _EB), lambda i: (i, 0, 0)),
        out_shape=jax.ShapeDtypeStruct((nb, 1, _EB), jnp.float32),
    )(u3, t3, w_t, mb1, mW2, mb2)
    return out.reshape(E)


def kernel(edge_index, edge_type, e_text_emb, node_emb, Wr1, Wroot1, b1,
           Wr2, Wroot2, b2, mW1, mb1, mW2, mb2):
    src = edge_index[0]
    dst = edge_index[1]
    et = edge_type

    key3 = (dst * R + et).astype(jnp.int32).reshape(NW, NB, BE)
    iy3 = (et * N + src).astype(jnp.int32).reshape(NW, NB, BE)
    dst3 = dst.astype(jnp.int32).reshape(NW, NB, BE)
    ip3 = (2 * src).astype(jnp.int32).reshape(NW, NB, BE)
    iq3 = (2 * dst + 1).astype(jnp.int32).reshape(NW, NB, BE)

    hpart = _hist_k(key3)
    invc = _inv_k(hpart)
    scale3 = _scale_k(key3, invc)
    iy4 = iy3.reshape(NW, 5, 25, BE)
    dst4 = dst3.reshape(NW, 5, 25, BE)
    scale4 = scale3.reshape(NW, 5, 25, BE)

    x0 = node_emb
    Y1 = _ymul_k(x0, Wr1)
    a1 = _agg_k(Y1, iy4, dst4, scale4)
    h1 = _dense_k(a1, x0, Wroot1, b1)

    Y2 = _ymul_k(h1, Wr2)
    a2 = _agg_k(Y2, iy4, dst4, scale4)
    h = _dense_k(a2, h1, Wroot2, b2)

    Wpq = jnp.concatenate([mW1[:128], mW1[128:256]], axis=1)
    PQ2 = _pq_k(h, Wpq).reshape(2 * N, 128)
    u = _gath_k(PQ2, ip3.reshape(NW, 5, 25, BE), iq3.reshape(NW, 5, 25, BE))
    logits = _mlp_k(u, e_text_emb, mW1[256:], mb1, mW2, mb2)
    return (logits, h)


# speedup vs baseline: 5.8535x; 1.2523x over previous
import functools
import jax
import jax.numpy as jnp
from jax import lax
from jax.experimental import pallas as pl
from jax.experimental.pallas import tpu as pltpu
from jax.experimental.pallas import tpu_sc as plsc

N = 10000
R = 8
E = 320000
NW = 32
EPW = E // NW
NB = 125
BE = 80
HB = 81920
NP = 10240

_mesh = plsc.VectorSubcoreMesh(core_axis_name="c", subcore_axis_name="s")


def _wid():
    return lax.axis_index("c") * 16 + lax.axis_index("s")


@functools.partial(
    pl.kernel,
    out_type=jax.ShapeDtypeStruct((2, HB), jnp.float32),
    mesh=_mesh,
    scratch_types=[
        pltpu.VMEM((NB, BE), jnp.int32),
        pltpu.VMEM((BE,), jnp.float32),
        pltpu.VMEM((5120,), jnp.float32),
        pltpu.VMEM_SHARED((HB,), jnp.float32),
    ],
)
def _hist_k(key_hbm, out_hbm, kb, ones, zb, hist):
    c = lax.axis_index("c")
    s = lax.axis_index("s")
    w = _wid()

    def fill_ones(i, _):
        ones[pl.ds(i * 16, 16)] = jnp.ones((16,), jnp.float32)
        return 0

    lax.fori_loop(0, BE // 16, fill_ones, 0)

    def fill_z(i, _):
        zb[pl.ds(i * 16, 16)] = jnp.zeros((16,), jnp.float32)
        return 0

    lax.fori_loop(0, 5120 // 16, fill_z, 0)
    pltpu.sync_copy(zb, hist.at[pl.ds(s * 5120, 5120)])
    plsc.subcore_barrier()

    pltpu.sync_copy(key_hbm.at[w], kb)

    def body(j, _):
        pltpu.sync_copy(ones, hist.at[kb.at[j]], add=True)
        return 0

    lax.fori_loop(0, NB, body, 0)
    plsc.subcore_barrier()
    pltpu.sync_copy(hist.at[pl.ds(s * 5120, 5120)],
                    out_hbm.at[c, pl.ds(s * 5120, 5120)])


@functools.partial(
    pl.kernel,
    out_type=jax.ShapeDtypeStruct((NW, NB, BE), jnp.float32),
    mesh=_mesh,
    scratch_types=[
        pltpu.VMEM((NB, BE), jnp.int32),
        pltpu.VMEM((NB, BE), jnp.float32),
        pltpu.SemaphoreType.DMA,
    ],
)
def _scale_k(key_hbm, invc_hbm, out_hbm, kb, sb, sem):
    w = _wid()
    pltpu.sync_copy(key_hbm.at[w], kb)

    def chunk(cc, _):
        j0 = cc * 25
        descs = []
        for b in range(25):
            descs.append(pltpu.async_copy(
                invc_hbm.at[kb.at[j0 + b]], sb.at[j0 + b], sem))
        for b in range(25):
            descs[b].wait()
        return 0

    lax.fori_loop(0, 5, chunk, 0)
    pltpu.sync_copy(sb, out_hbm.at[w])


@functools.partial(
    pl.kernel,
    out_type=jax.ShapeDtypeStruct((2, NP, 128), jnp.float32),
    mesh=_mesh,
    scratch_types=[
        pltpu.VMEM((25, BE), jnp.int32),
        pltpu.VMEM((25, BE), jnp.float32),
        pltpu.VMEM((2, BE), jnp.int32),
        pltpu.VMEM((2, BE), jnp.int32),
        pltpu.VMEM((2, BE, 128), jnp.float32),
        pltpu.VMEM((16, 128), jnp.float32),
        pltpu.VMEM_SHARED((NP, 128), jnp.float32),
        pltpu.SemaphoreType.DMA,
    ],
)
def _agg_k(y_hbm, pk_hbm, sc_hbm, out_hbm,
           pkb, sb, iyblk, dstblk, rb, zb, acc, sem):
    c = lax.axis_index("c")
    s = lax.axis_index("s")
    w = _wid()

    def fill_z(i, _):
        row = i // 8
        f = i % 8
        zb[row, pl.ds(f * 16, 16)] = jnp.zeros((16,), jnp.float32)
        return 0

    lax.fori_loop(0, 16 * 8, fill_z, 0)

    def zcopy(i, _):
        pltpu.sync_copy(zb, acc.at[pl.ds(s * 640 + i * 16, 16)])
        return 0

    lax.fori_loop(0, 40, zcopy, 0)
    plsc.subcore_barrier()

    def unpack(j, buf):
        for i in range(BE // 16):
            v = pkb[j, pl.ds(i * 16, 16)]
            iyblk[buf, pl.ds(i * 16, 16)] = v & 0x1FFFF
            dstblk[buf, pl.ds(i * 16, 16)] = lax.shift_right_logical(v, 17)

    def issue(buf):
        return pltpu.async_copy(y_hbm.at[iyblk.at[buf]], rb.at[buf], sem)

    def wait_g():
        pltpu.make_async_copy(y_hbm.at[iyblk.at[0]], rb.at[0], sem).wait()

    def scale_scatter(j, buf):
        def escale(i, _):
            sc16 = sb[j, pl.ds(i * 16, 16)]
            for ee in range(16):
                e = i * 16 + ee
                sc = sc16[ee]
                for f in range(8):
                    rb[buf, e, pl.ds(f * 16, 16)] = (
                        rb[buf, e, pl.ds(f * 16, 16)] * sc)
            return 0

        lax.fori_loop(0, BE // 16, escale, 0)
        pltpu.sync_copy(rb.at[buf], acc.at[dstblk.at[buf]], add=True)

    def chunk(cc, _):
        pltpu.sync_copy(pk_hbm.at[w, cc], pkb)
        pltpu.sync_copy(sc_hbm.at[w, cc], sb)
        unpack(0, 0)
        issue(0)
        for b in range(1, 25):
            unpack(b, b % 2)
            issue(b % 2)
            wait_g()
            scale_scatter(b - 1, (b - 1) % 2)
        wait_g()
        scale_scatter(24, 0)
        return 0

    lax.fori_loop(0, 5, chunk, 0)
    plsc.subcore_barrier()
    pltpu.sync_copy(acc.at[pl.ds(s * 640, 640)],
                    out_hbm.at[c, pl.ds(s * 640, 640)])


@functools.partial(
    pl.kernel,
    out_type=jax.ShapeDtypeStruct((E, 128), jnp.float32),
    mesh=_mesh,
    scratch_types=[
        pltpu.VMEM((25, BE), jnp.int32),
        pltpu.VMEM((25, BE), jnp.int32),
        pltpu.VMEM((5, BE, 128), jnp.float32),
        pltpu.VMEM((5, BE, 128), jnp.float32),
        pltpu.SemaphoreType.DMA,
        pltpu.SemaphoreType.DMA,
        pltpu.SemaphoreType.DMA,
    ],
)
def _gath_k(pq_hbm, ip_hbm, iq_hbm, u_hbm,
            ipb, iqb, pb, qb, sp, sq, sw):
    w = _wid()

    def chunk(cc, _):
        pltpu.sync_copy(ip_hbm.at[w, cc], ipb)
        pltpu.sync_copy(iq_hbm.at[w, cc], iqb)

        def sub(dd, _):
            j0 = dd * 5
            cps = []
            cqs = []
            for b in range(5):
                cps.append(pltpu.async_copy(
                    pq_hbm.at[ipb.at[j0 + b]], pb.at[b], sp))
                cqs.append(pltpu.async_copy(
                    pq_hbm.at[iqb.at[j0 + b]], qb.at[b], sq))
            cws = []
            for b in range(5):
                cps[b].wait()
                cqs[b].wait()

                def eadd(e, _):
                    for f in range(8):
                        pb[b, e, pl.ds(f * 16, 16)] = (
                            pb[b, e, pl.ds(f * 16, 16)]
                            + qb[b, e, pl.ds(f * 16, 16)])
                    return 0

                lax.fori_loop(0, BE, eadd, 0)
                base = w * EPW + (cc * 25 + j0 + b) * BE
                cws.append(pltpu.async_copy(
                    pb.at[b], u_hbm.at[pl.ds(base, BE)], sw))
            for b in range(5):
                cws[b].wait()
            return 0

        lax.fori_loop(0, 5, sub, 0)
        return 0

    lax.fori_loop(0, 5, chunk, 0)


def _inv_body(p_ref, o_ref):
    o_ref[...] = 1.0 / jnp.maximum(p_ref[0] + p_ref[1], 1.0)


def _inv_k(hpart):
    h3 = hpart.reshape(2, 640, 128)
    out = pl.pallas_call(
        _inv_body,
        grid=(5,),
        in_specs=[pl.BlockSpec((2, 128, 128), lambda i: (0, i, 0))],
        out_specs=pl.BlockSpec((128, 128), lambda i: (i, 0)),
        out_shape=jax.ShapeDtypeStruct((640, 128), jnp.float32),
    )(h3)
    return out.reshape(HB)


def _ymul_body(x_ref, w_ref, o_ref):
    o_ref[0] = jnp.dot(x_ref[...], w_ref[0],
                       preferred_element_type=jnp.float32)


def _ymul_k(x, Wr):
    out = pl.pallas_call(
        _ymul_body,
        grid=(R, 5),
        in_specs=[
            pl.BlockSpec((2000, 128), lambda r, i: (i, 0)),
            pl.BlockSpec((1, 128, 128), lambda r, i: (r, 0, 0)),
        ],
        out_specs=pl.BlockSpec((1, 2000, 128), lambda r, i: (r, i, 0)),
        out_shape=jax.ShapeDtypeStruct((R, N, 128), jnp.float32),
    )(x, Wr)
    return out.reshape(R * N, 128)


def _dense_body(a_ref, x_ref, wr_ref, b_ref, o_ref):
    o_ref[...] = jnp.maximum(
        a_ref[0] + a_ref[1]
        + jnp.dot(x_ref[...], wr_ref[...], preferred_element_type=jnp.float32)
        + b_ref[...], 0.0)


def _dense_k(apart, x, Wroot, b):
    return pl.pallas_call(
        _dense_body,
        grid=(5,),
        in_specs=[
            pl.BlockSpec((2, 2000, 128), lambda i: (0, i, 0)),
            pl.BlockSpec((2000, 128), lambda i: (i, 0)),
            pl.BlockSpec((128, 128), lambda i: (0, 0)),
            pl.BlockSpec((128,), lambda i: (0,)),
        ],
        out_specs=pl.BlockSpec((2000, 128), lambda i: (i, 0)),
        out_shape=jax.ShapeDtypeStruct((N, 128), jnp.float32),
    )(apart, x, Wroot, b)


def _pq_body(x_ref, w_ref, o_ref):
    o_ref[...] = jnp.dot(x_ref[...], w_ref[...],
                         preferred_element_type=jnp.float32)


def _pq_k(h, Wpq):
    return pl.pallas_call(
        _pq_body,
        grid=(5,),
        in_specs=[
            pl.BlockSpec((2000, 128), lambda i: (i, 0)),
            pl.BlockSpec((128, 256), lambda i: (0, 0)),
        ],
        out_specs=pl.BlockSpec((2000, 256), lambda i: (i, 0)),
        out_shape=jax.ShapeDtypeStruct((N, 256), jnp.float32),
    )(h, Wpq)


_EB = 2000


def _mlp_body(u_ref, t_ref, w_ref, mb1_ref, w2_ref, mb2_ref, o_ref):
    u = u_ref[0]
    t = t_ref[0]
    hid = jnp.maximum(u + jnp.dot(t, w_ref[...],
                                  preferred_element_type=jnp.float32)
                      + mb1_ref[...], 0.0)
    lg = jnp.dot(hid, w2_ref[...], preferred_element_type=jnp.float32)
    o_ref[0, 0, :] = lg[:, 0] + mb2_ref[0]


def _mlp_k(u, etext, w_t, mb1, mW2, mb2):
    nb = E // _EB
    u3 = u.reshape(nb, _EB, 128)
    t3 = etext.reshape(nb, _EB, 128)
    out = pl.pallas_call(
        _mlp_body,
        grid=(nb,),
        in_specs=[
            pl.BlockSpec((1, _EB, 128), lambda i: (i, 0, 0)),
            pl.BlockSpec((1, _EB, 128), lambda i: (i, 0, 0)),
            pl.BlockSpec((128, 128), lambda i: (0, 0)),
            pl.BlockSpec((128,), lambda i: (0,)),
            pl.BlockSpec((128, 1), lambda i: (0, 0)),
            pl.BlockSpec((1,), lambda i: (0,)),
        ],
        out_specs=pl.BlockSpec((1, 1, _EB), lambda i: (i, 0, 0)),
        out_shape=jax.ShapeDtypeStruct((nb, 1, _EB), jnp.float32),
    )(u3, t3, w_t, mb1, mW2, mb2)
    return out.reshape(E)


def kernel(edge_index, edge_type, e_text_emb, node_emb, Wr1, Wroot1, b1,
           Wr2, Wroot2, b2, mW1, mb1, mW2, mb2):
    src = edge_index[0]
    dst = edge_index[1]
    et = edge_type

    key3 = (dst * R + et).astype(jnp.int32).reshape(NW, NB, BE)
    iy = (et * N + src).astype(jnp.int32)
    pk3 = (iy | (dst.astype(jnp.int32) << 17)).reshape(NW, NB, BE)
    ip3 = (2 * src).astype(jnp.int32).reshape(NW, NB, BE)
    iq3 = (2 * dst + 1).astype(jnp.int32).reshape(NW, NB, BE)

    hpart = _hist_k(key3)
    invc = _inv_k(hpart)
    scale3 = _scale_k(key3, invc)


    x0 = node_emb
    Y1 = _ymul_k(x0, Wr1)
    pk4 = pk3.reshape(NW, 5, 25, BE)
    sc4 = scale3.reshape(NW, 5, 25, BE)
    a1 = _agg_k(Y1, pk4, sc4)
    h1 = _dense_k(a1, x0, Wroot1, b1)

    Y2 = _ymul_k(h1, Wr2)
    a2 = _agg_k(Y2, pk4, sc4)
    h = _dense_k(a2, h1, Wroot2, b2)

    Wpq = jnp.concatenate([mW1[:128], mW1[128:256]], axis=1)
    PQ2 = _pq_k(h, Wpq).reshape(2 * N, 128)
    u = _gath_k(PQ2, ip3.reshape(NW, 5, 25, BE), iq3.reshape(NW, 5, 25, BE))
    logits = _mlp_k(u, e_text_emb, mW1[256:], mb1, mW2, mb2)
    return (logits, h)


# speedup vs baseline: 6.4354x; 1.0994x over previous
import functools
import jax
import jax.numpy as jnp
from jax import lax
from jax.experimental import pallas as pl
from jax.experimental.pallas import tpu as pltpu
from jax.experimental.pallas import tpu_sc as plsc

N = 10000
R = 8
E = 320000
NW = 32
EPW = E // NW
NB = 125
BE = 80
HB = 81920
NP = 10240

_mesh = plsc.VectorSubcoreMesh(core_axis_name="c", subcore_axis_name="s")


def _wid():
    return lax.axis_index("c") * 16 + lax.axis_index("s")


@functools.partial(
    pl.kernel,
    out_type=jax.ShapeDtypeStruct((2, HB), jnp.float32),
    mesh=_mesh,
    scratch_types=[
        pltpu.VMEM((NB, BE), jnp.int32),
        pltpu.VMEM((BE,), jnp.float32),
        pltpu.VMEM((5120,), jnp.float32),
        pltpu.VMEM_SHARED((HB,), jnp.float32),
    ],
)
def _hist_k(key_hbm, out_hbm, kb, ones, zb, hist):
    c = lax.axis_index("c")
    s = lax.axis_index("s")
    w = _wid()

    def fill_ones(i, _):
        ones[pl.ds(i * 16, 16)] = jnp.ones((16,), jnp.float32)
        return 0

    lax.fori_loop(0, BE // 16, fill_ones, 0)

    def fill_z(i, _):
        zb[pl.ds(i * 16, 16)] = jnp.zeros((16,), jnp.float32)
        return 0

    lax.fori_loop(0, 5120 // 16, fill_z, 0)
    pltpu.sync_copy(zb, hist.at[pl.ds(s * 5120, 5120)])
    plsc.subcore_barrier()

    pltpu.sync_copy(key_hbm.at[w], kb)

    def body(j, _):
        pltpu.sync_copy(ones, hist.at[kb.at[j]], add=True)
        return 0

    lax.fori_loop(0, NB, body, 0)
    plsc.subcore_barrier()
    pltpu.sync_copy(hist.at[pl.ds(s * 5120, 5120)],
                    out_hbm.at[c, pl.ds(s * 5120, 5120)])


@functools.partial(
    pl.kernel,
    out_type=jax.ShapeDtypeStruct((NW, NB, BE), jnp.float32),
    mesh=_mesh,
    scratch_types=[
        pltpu.VMEM((NB, BE), jnp.int32),
        pltpu.VMEM((NB, BE), jnp.float32),
        pltpu.SemaphoreType.DMA,
    ],
)
def _scale_k(key_hbm, invc_hbm, out_hbm, kb, sb, sem):
    w = _wid()
    pltpu.sync_copy(key_hbm.at[w], kb)

    def chunk(cc, _):
        j0 = cc * 25
        descs = []
        for b in range(25):
            descs.append(pltpu.async_copy(
                invc_hbm.at[kb.at[j0 + b]], sb.at[j0 + b], sem))
        for b in range(25):
            descs[b].wait()
        return 0

    lax.fori_loop(0, 5, chunk, 0)
    pltpu.sync_copy(sb, out_hbm.at[w])


@functools.partial(
    pl.kernel,
    out_type=jax.ShapeDtypeStruct((2, NP, 128), jnp.float32),
    mesh=_mesh,
    scratch_types=[
        pltpu.VMEM((25, BE), jnp.int32),
        pltpu.VMEM((25, BE), jnp.float32),
        pltpu.VMEM((3, BE), jnp.int32),
        pltpu.VMEM((3, BE), jnp.int32),
        pltpu.VMEM((3, BE, 128), jnp.float32),
        pltpu.VMEM((16, 128), jnp.float32),
        pltpu.VMEM_SHARED((NP, 128), jnp.float32),
        pltpu.SemaphoreType.DMA,
        pltpu.SemaphoreType.DMA,
    ],
)
def _agg_k(y_hbm, pk_hbm, sc_hbm, out_hbm,
           pkb, sb, iyblk, dstblk, rb, zb, acc, sem, ssc):
    c = lax.axis_index("c")
    s = lax.axis_index("s")
    w = _wid()

    def fill_z(i, _):
        row = i // 8
        f = i % 8
        zb[row, pl.ds(f * 16, 16)] = jnp.zeros((16,), jnp.float32)
        return 0

    lax.fori_loop(0, 16 * 8, fill_z, 0)

    def zcopy(i, _):
        pltpu.sync_copy(zb, acc.at[pl.ds(s * 640 + i * 16, 16)])
        return 0

    lax.fori_loop(0, 40, zcopy, 0)
    plsc.subcore_barrier()

    def unpack(j, buf):
        for i in range(BE // 16):
            v = pkb[j, pl.ds(i * 16, 16)]
            iyblk[buf, pl.ds(i * 16, 16)] = v & 0x1FFFF
            dstblk[buf, pl.ds(i * 16, 16)] = lax.shift_right_logical(v, 17)

    def issue(buf):
        return pltpu.async_copy(y_hbm.at[iyblk.at[buf]], rb.at[buf], sem)

    def wait_g():
        pltpu.make_async_copy(y_hbm.at[iyblk.at[0]], rb.at[0], sem).wait()

    def scale_scatter(j, buf):
        def escale(i, _):
            sc16 = sb[j, pl.ds(i * 16, 16)]
            for ee in range(16):
                e = i * 16 + ee
                sc = sc16[ee]
                for f in range(8):
                    rb[buf, e, pl.ds(f * 16, 16)] = (
                        rb[buf, e, pl.ds(f * 16, 16)] * sc)
            return 0

        lax.fori_loop(0, BE // 16, escale, 0)
        pltpu.async_copy(rb.at[buf], acc.at[dstblk.at[buf]], ssc, add=True)

    def wait_sc():
        pltpu.make_async_copy(rb.at[0], acc.at[dstblk.at[0]], ssc).wait()

    def chunk(cc, _):
        pltpu.sync_copy(pk_hbm.at[w, cc], pkb)
        pltpu.sync_copy(sc_hbm.at[w, cc], sb)
        unpack(0, 0)
        issue(0)
        for b in range(1, 25):
            if b >= 3:
                wait_sc()
            unpack(b, b % 3)
            issue(b % 3)
            wait_g()
            scale_scatter(b - 1, (b - 1) % 3)
        wait_g()
        scale_scatter(24, 0)
        for _i in range(3):
            wait_sc()
        return 0

    lax.fori_loop(0, 5, chunk, 0)
    plsc.subcore_barrier()
    pltpu.sync_copy(acc.at[pl.ds(s * 640, 640)],
                    out_hbm.at[c, pl.ds(s * 640, 640)])


def _make_gath(nchunk, off, ne):
    @functools.partial(
        pl.kernel,
        out_type=jax.ShapeDtypeStruct((NW * ne, 128), jnp.float32),
        mesh=_mesh,
        scratch_types=[
            pltpu.VMEM((25, BE), jnp.int32),
            pltpu.VMEM((25, BE), jnp.int32),
            pltpu.VMEM((5, BE, 128), jnp.float32),
            pltpu.VMEM((5, BE, 128), jnp.float32),
            pltpu.SemaphoreType.DMA,
            pltpu.SemaphoreType.DMA,
            pltpu.SemaphoreType.DMA,
        ],
    )
    def _gath_k(pq_hbm, ip_hbm, iq_hbm, u_hbm,
                ipb, iqb, pb, qb, sp, sq, sw):
        w = _wid()

        def chunk(cc, _):
            pltpu.sync_copy(ip_hbm.at[w, off + cc], ipb)
            pltpu.sync_copy(iq_hbm.at[w, off + cc], iqb)

            def sub(dd, _):
                j0 = dd * 5
                cps = []
                cqs = []
                for b in range(5):
                    cps.append(pltpu.async_copy(
                        pq_hbm.at[ipb.at[j0 + b]], pb.at[b], sp))
                    cqs.append(pltpu.async_copy(
                        pq_hbm.at[iqb.at[j0 + b]], qb.at[b], sq))
                cws = []
                for b in range(5):
                    cps[b].wait()
                    cqs[b].wait()

                    def eadd(e, _):
                        for f in range(8):
                            pb[b, e, pl.ds(f * 16, 16)] = (
                                pb[b, e, pl.ds(f * 16, 16)]
                                + qb[b, e, pl.ds(f * 16, 16)])
                        return 0

                    lax.fori_loop(0, BE, eadd, 0)
                    base = w * ne + (cc * 25 + j0 + b) * BE
                    cws.append(pltpu.async_copy(
                        pb.at[b], u_hbm.at[pl.ds(base, BE)], sw))
                for b in range(5):
                    cws[b].wait()
                return 0

            lax.fori_loop(0, 5, sub, 0)
            return 0

        lax.fori_loop(0, nchunk, chunk, 0)

    return _gath_k


_gath_a = _make_gath(3, 0, 6000)
_gath_b = _make_gath(2, 3, 4000)


def _inv_body(p_ref, o_ref):
    o_ref[...] = 1.0 / jnp.maximum(p_ref[0] + p_ref[1], 1.0)


def _inv_k(hpart):
    h3 = hpart.reshape(2, 640, 128)
    out = pl.pallas_call(
        _inv_body,
        grid=(5,),
        in_specs=[pl.BlockSpec((2, 128, 128), lambda i: (0, i, 0))],
        out_specs=pl.BlockSpec((128, 128), lambda i: (i, 0)),
        out_shape=jax.ShapeDtypeStruct((640, 128), jnp.float32),
    )(h3)
    return out.reshape(HB)


def _ymul_body(x_ref, w_ref, o_ref):
    o_ref[0] = jnp.dot(x_ref[...], w_ref[0],
                       preferred_element_type=jnp.float32)


def _ymul_k(x, Wr):
    out = pl.pallas_call(
        _ymul_body,
        grid=(R, 5),
        in_specs=[
            pl.BlockSpec((2000, 128), lambda r, i: (i, 0)),
            pl.BlockSpec((1, 128, 128), lambda r, i: (r, 0, 0)),
        ],
        out_specs=pl.BlockSpec((1, 2000, 128), lambda r, i: (r, i, 0)),
        out_shape=jax.ShapeDtypeStruct((R, N, 128), jnp.float32),
    )(x, Wr)
    return out.reshape(R * N, 128)


def _dense_body(a_ref, x_ref, wr_ref, b_ref, o_ref):
    o_ref[...] = jnp.maximum(
        a_ref[0] + a_ref[1]
        + jnp.dot(x_ref[...], wr_ref[...], preferred_element_type=jnp.float32)
        + b_ref[...], 0.0)


def _dense_k(apart, x, Wroot, b):
    return pl.pallas_call(
        _dense_body,
        grid=(5,),
        in_specs=[
            pl.BlockSpec((2, 2000, 128), lambda i: (0, i, 0)),
            pl.BlockSpec((2000, 128), lambda i: (i, 0)),
            pl.BlockSpec((128, 128), lambda i: (0, 0)),
            pl.BlockSpec((128,), lambda i: (0,)),
        ],
        out_specs=pl.BlockSpec((2000, 128), lambda i: (i, 0)),
        out_shape=jax.ShapeDtypeStruct((N, 128), jnp.float32),
    )(apart, x, Wroot, b)


def _pq_body(x_ref, w_ref, o_ref):
    o_ref[...] = jnp.dot(x_ref[...], w_ref[...],
                         preferred_element_type=jnp.float32)


def _pq_k(h, Wpq):
    return pl.pallas_call(
        _pq_body,
        grid=(5,),
        in_specs=[
            pl.BlockSpec((2000, 128), lambda i: (i, 0)),
            pl.BlockSpec((128, 256), lambda i: (0, 0)),
        ],
        out_specs=pl.BlockSpec((2000, 256), lambda i: (i, 0)),
        out_shape=jax.ShapeDtypeStruct((N, 256), jnp.float32),
    )(h, Wpq)


_EB = 2000


def _mlp_body(u_ref, t_ref, w_ref, mb1_ref, w2_ref, mb2_ref, o_ref):
    u = u_ref[0]
    t = t_ref[0]
    hid = jnp.maximum(u + jnp.dot(t, w_ref[...],
                                  preferred_element_type=jnp.float32)
                      + mb1_ref[...], 0.0)
    lg = jnp.dot(hid, w2_ref[...], preferred_element_type=jnp.float32)
    o_ref[0, 0, :] = lg[:, 0] + mb2_ref[0]


def _mlp_part(u, t3, w_t, mb1, mW2, mb2, kpt, koff):
    nb = NW * kpt
    u3 = u.reshape(nb, _EB, 128)
    out = pl.pallas_call(
        _mlp_body,
        grid=(nb,),
        in_specs=[
            pl.BlockSpec((1, _EB, 128), lambda i: (i, 0, 0)),
            pl.BlockSpec((1, _EB, 128),
                         lambda i: (i // kpt * 5 + koff + i % kpt, 0, 0)),
            pl.BlockSpec((128, 128), lambda i: (0, 0)),
            pl.BlockSpec((128,), lambda i: (0,)),
            pl.BlockSpec((128, 1), lambda i: (0, 0)),
            pl.BlockSpec((1,), lambda i: (0,)),
        ],
        out_specs=pl.BlockSpec((1, 1, _EB), lambda i: (i, 0, 0)),
        out_shape=jax.ShapeDtypeStruct((nb, 1, _EB), jnp.float32),
    )(u3, t3, w_t, mb1, mW2, mb2)
    return out.reshape(NW, kpt * _EB)


def kernel(edge_index, edge_type, e_text_emb, node_emb, Wr1, Wroot1, b1,
           Wr2, Wroot2, b2, mW1, mb1, mW2, mb2):
    src = edge_index[0]
    dst = edge_index[1]
    et = edge_type

    key3 = (dst * R + et).astype(jnp.int32).reshape(NW, NB, BE)
    iy = (et * N + src).astype(jnp.int32)
    pk3 = (iy | (dst.astype(jnp.int32) << 17)).reshape(NW, NB, BE)
    ip3 = (2 * src).astype(jnp.int32).reshape(NW, NB, BE)
    iq3 = (2 * dst + 1).astype(jnp.int32).reshape(NW, NB, BE)

    hpart = _hist_k(key3)
    invc = _inv_k(hpart)
    scale3 = _scale_k(key3, invc)


    x0 = node_emb
    Y1 = _ymul_k(x0, Wr1)
    pk4 = pk3.reshape(NW, 5, 25, BE)
    sc4 = scale3.reshape(NW, 5, 25, BE)
    a1 = _agg_k(Y1, pk4, sc4)
    h1 = _dense_k(a1, x0, Wroot1, b1)

    Y2 = _ymul_k(h1, Wr2)
    a2 = _agg_k(Y2, pk4, sc4)
    h = _dense_k(a2, h1, Wroot2, b2)

    Wpq = jnp.concatenate([mW1[:128], mW1[128:256]], axis=1)
    PQ2 = _pq_k(h, Wpq).reshape(2 * N, 128)
    ip4 = ip3.reshape(NW, 5, 25, BE)
    iq4 = iq3.reshape(NW, 5, 25, BE)
    t3 = e_text_emb.reshape(NW * 5, _EB, 128)
    w_t = mW1[256:]
    ua = _gath_a(PQ2, ip4, iq4)
    la = _mlp_part(ua, t3, w_t, mb1, mW2, mb2, 3, 0)
    ub = _gath_b(PQ2, ip4, iq4)
    lb = _mlp_part(ub, t3, w_t, mb1, mW2, mb2, 2, 3)
    logits = jnp.concatenate([la, lb], axis=1).reshape(E)
    return (logits, h)


# speedup vs baseline: 6.6262x; 1.0296x over previous
import functools
import jax
import jax.numpy as jnp
from jax import lax
from jax.experimental import pallas as pl
from jax.experimental.pallas import tpu as pltpu
from jax.experimental.pallas import tpu_sc as plsc

N = 10000
R = 8
E = 320000
NW = 32
EPW = E // NW
NB = 125
BE = 80
HB = 81920
NP = 10240

_mesh = plsc.VectorSubcoreMesh(core_axis_name="c", subcore_axis_name="s")


def _wid():
    return lax.axis_index("c") * 16 + lax.axis_index("s")


@functools.partial(
    pl.kernel,
    out_type=jax.ShapeDtypeStruct((2, HB), jnp.float32),
    mesh=_mesh,
    scratch_types=[
        pltpu.VMEM((NB, BE), jnp.int32),
        pltpu.VMEM((BE,), jnp.float32),
        pltpu.VMEM((5120,), jnp.float32),
        pltpu.VMEM_SHARED((HB,), jnp.float32),
    ],
)
def _hist_k(key_hbm, out_hbm, kb, ones, zb, hist):
    c = lax.axis_index("c")
    s = lax.axis_index("s")
    w = _wid()

    def fill_ones(i, _):
        ones[pl.ds(i * 16, 16)] = jnp.ones((16,), jnp.float32)
        return 0

    lax.fori_loop(0, BE // 16, fill_ones, 0)

    def fill_z(i, _):
        zb[pl.ds(i * 16, 16)] = jnp.zeros((16,), jnp.float32)
        return 0

    lax.fori_loop(0, 5120 // 16, fill_z, 0)
    pltpu.sync_copy(zb, hist.at[pl.ds(s * 5120, 5120)])
    plsc.subcore_barrier()

    pltpu.sync_copy(key_hbm.at[w], kb)

    def body(j, _):
        pltpu.sync_copy(ones, hist.at[kb.at[j]], add=True)
        return 0

    lax.fori_loop(0, NB, body, 0)
    plsc.subcore_barrier()
    pltpu.sync_copy(hist.at[pl.ds(s * 5120, 5120)],
                    out_hbm.at[c, pl.ds(s * 5120, 5120)])


@functools.partial(
    pl.kernel,
    out_type=jax.ShapeDtypeStruct((NW, NB, BE), jnp.float32),
    mesh=_mesh,
    scratch_types=[
        pltpu.VMEM((NB, BE), jnp.int32),
        pltpu.VMEM((NB, BE), jnp.float32),
        pltpu.SemaphoreType.DMA,
    ],
)
def _scale_k(key_hbm, invc_hbm, out_hbm, kb, sb, sem):
    w = _wid()
    pltpu.sync_copy(key_hbm.at[w], kb)

    def chunk(cc, _):
        j0 = cc * 25
        descs = []
        for b in range(25):
            descs.append(pltpu.async_copy(
                invc_hbm.at[kb.at[j0 + b]], sb.at[j0 + b], sem))
        for b in range(25):
            descs[b].wait()
        return 0

    lax.fori_loop(0, 5, chunk, 0)
    pltpu.sync_copy(sb, out_hbm.at[w])


@functools.partial(
    pl.kernel,
    out_type=jax.ShapeDtypeStruct((2, NP, 128), jnp.float32),
    mesh=_mesh,
    scratch_types=[
        pltpu.VMEM((25, BE), jnp.int32),
        pltpu.VMEM((25, BE), jnp.float32),
        pltpu.VMEM((3, BE), jnp.int32),
        pltpu.VMEM((3, BE), jnp.int32),
        pltpu.VMEM((3, BE, 128), jnp.float32),
        pltpu.VMEM((16, 128), jnp.float32),
        pltpu.VMEM_SHARED((NP, 128), jnp.float32),
        pltpu.SemaphoreType.DMA,
        pltpu.SemaphoreType.DMA,
    ],
)
def _agg_k(y_hbm, pk_hbm, sc_hbm, out_hbm,
           pkb, sb, iyblk, dstblk, rb, zb, acc, sem, ssc):
    c = lax.axis_index("c")
    s = lax.axis_index("s")
    w = _wid()

    def fill_z(i, _):
        row = i // 8
        f = i % 8
        zb[row, pl.ds(f * 16, 16)] = jnp.zeros((16,), jnp.float32)
        return 0

    lax.fori_loop(0, 16 * 8, fill_z, 0)

    def zcopy(i, _):
        pltpu.sync_copy(zb, acc.at[pl.ds(s * 640 + i * 16, 16)])
        return 0

    lax.fori_loop(0, 40, zcopy, 0)
    plsc.subcore_barrier()

    def unpack(j, buf):
        for i in range(BE // 16):
            v = pkb[j, pl.ds(i * 16, 16)]
            iyblk[buf, pl.ds(i * 16, 16)] = v & 0x1FFFF
            dstblk[buf, pl.ds(i * 16, 16)] = lax.shift_right_logical(v, 17)

    def issue(buf):
        return pltpu.async_copy(y_hbm.at[iyblk.at[buf]], rb.at[buf], sem)

    def wait_g():
        pltpu.make_async_copy(y_hbm.at[iyblk.at[0]], rb.at[0], sem).wait()

    def scale_scatter(j, buf):
        def escale(i, _):
            sc16 = sb[j, pl.ds(i * 16, 16)]
            for ee in range(16):
                e = i * 16 + ee
                sc = sc16[ee]
                for f in range(8):
                    rb[buf, e, pl.ds(f * 16, 16)] = (
                        rb[buf, e, pl.ds(f * 16, 16)] * sc)
            return 0

        lax.fori_loop(0, BE // 16, escale, 0)
        pltpu.async_copy(rb.at[buf], acc.at[dstblk.at[buf]], ssc, add=True)

    def wait_sc():
        pltpu.make_async_copy(rb.at[0], acc.at[dstblk.at[0]], ssc).wait()

    def chunk(cc, _):
        pltpu.sync_copy(pk_hbm.at[w, cc], pkb)
        pltpu.sync_copy(sc_hbm.at[w, cc], sb)
        unpack(0, 0)
        issue(0)
        for b in range(1, 25):
            if b >= 3:
                wait_sc()
            unpack(b, b % 3)
            issue(b % 3)
            wait_g()
            scale_scatter(b - 1, (b - 1) % 3)
        wait_g()
        scale_scatter(24, 0)
        for _i in range(3):
            wait_sc()
        return 0

    lax.fori_loop(0, 5, chunk, 0)
    plsc.subcore_barrier()
    pltpu.sync_copy(acc.at[pl.ds(s * 640, 640)],
                    out_hbm.at[c, pl.ds(s * 640, 640)])


def _make_gath(nchunk, off, ne):
    @functools.partial(
        pl.kernel,
        out_type=jax.ShapeDtypeStruct((NW * ne, 128), jnp.float32),
        mesh=_mesh,
        scratch_types=[
            pltpu.VMEM((25, BE), jnp.int32),
            pltpu.VMEM((25, BE), jnp.int32),
            pltpu.VMEM((5, BE, 128), jnp.float32),
            pltpu.VMEM((5, BE, 128), jnp.float32),
            pltpu.SemaphoreType.DMA,
            pltpu.SemaphoreType.DMA,
            pltpu.SemaphoreType.DMA,
        ],
    )
    def _gath_k(pq_hbm, ip_hbm, iq_hbm, u_hbm,
                ipb, iqb, pb, qb, sp, sq, sw):
        w = _wid()

        def chunk(cc, _):
            pltpu.sync_copy(ip_hbm.at[w, off + cc], ipb)
            pltpu.sync_copy(iq_hbm.at[w, off + cc], iqb)

            def sub(dd, _):
                j0 = dd * 5
                cps = []
                cqs = []
                for b in range(5):
                    cps.append(pltpu.async_copy(
                        pq_hbm.at[ipb.at[j0 + b]], pb.at[b], sp))
                    cqs.append(pltpu.async_copy(
                        pq_hbm.at[iqb.at[j0 + b]], qb.at[b], sq))
                cws = []
                for b in range(5):
                    cps[b].wait()
                    cqs[b].wait()

                    def eadd(e, _):
                        for f in range(8):
                            pb[b, e, pl.ds(f * 16, 16)] = (
                                pb[b, e, pl.ds(f * 16, 16)]
                                + qb[b, e, pl.ds(f * 16, 16)])
                        return 0

                    lax.fori_loop(0, BE, eadd, 0)
                    base = w * ne + (cc * 25 + j0 + b) * BE
                    cws.append(pltpu.async_copy(
                        pb.at[b], u_hbm.at[pl.ds(base, BE)], sw))
                for b in range(5):
                    cws[b].wait()
                return 0

            lax.fori_loop(0, 5, sub, 0)
            return 0

        lax.fori_loop(0, nchunk, chunk, 0)

    return _gath_k


_gath_p = [_make_gath(1, k, 2000) for k in range(5)]


def _inv_body(p_ref, o_ref):
    o_ref[...] = 1.0 / jnp.maximum(p_ref[0] + p_ref[1], 1.0)


def _inv_k(hpart):
    h3 = hpart.reshape(2, 640, 128)
    out = pl.pallas_call(
        _inv_body,
        grid=(5,),
        in_specs=[pl.BlockSpec((2, 128, 128), lambda i: (0, i, 0))],
        out_specs=pl.BlockSpec((128, 128), lambda i: (i, 0)),
        out_shape=jax.ShapeDtypeStruct((640, 128), jnp.float32),
    )(h3)
    return out.reshape(HB)


def _ymul_body(x_ref, w_ref, o_ref):
    o_ref[0] = jnp.dot(x_ref[...], w_ref[0],
                       preferred_element_type=jnp.float32)


def _ymul_k(x, Wr):
    out = pl.pallas_call(
        _ymul_body,
        grid=(R, 5),
        in_specs=[
            pl.BlockSpec((2000, 128), lambda r, i: (i, 0)),
            pl.BlockSpec((1, 128, 128), lambda r, i: (r, 0, 0)),
        ],
        out_specs=pl.BlockSpec((1, 2000, 128), lambda r, i: (r, i, 0)),
        out_shape=jax.ShapeDtypeStruct((R, N, 128), jnp.float32),
    )(x, Wr)
    return out.reshape(R * N, 128)


def _dense_body(a_ref, x_ref, wr_ref, b_ref, o_ref):
    o_ref[...] = jnp.maximum(
        a_ref[0] + a_ref[1]
        + jnp.dot(x_ref[...], wr_ref[...], preferred_element_type=jnp.float32)
        + b_ref[...], 0.0)


def _dense_k(apart, x, Wroot, b):
    return pl.pallas_call(
        _dense_body,
        grid=(5,),
        in_specs=[
            pl.BlockSpec((2, 2000, 128), lambda i: (0, i, 0)),
            pl.BlockSpec((2000, 128), lambda i: (i, 0)),
            pl.BlockSpec((128, 128), lambda i: (0, 0)),
            pl.BlockSpec((128,), lambda i: (0,)),
        ],
        out_specs=pl.BlockSpec((2000, 128), lambda i: (i, 0)),
        out_shape=jax.ShapeDtypeStruct((N, 128), jnp.float32),
    )(apart, x, Wroot, b)


def _pq_body(x_ref, w_ref, o_ref):
    o_ref[...] = jnp.dot(x_ref[...], w_ref[...],
                         preferred_element_type=jnp.float32)


def _pq_k(h, Wpq):
    return pl.pallas_call(
        _pq_body,
        grid=(5,),
        in_specs=[
            pl.BlockSpec((2000, 128), lambda i: (i, 0)),
            pl.BlockSpec((128, 256), lambda i: (0, 0)),
        ],
        out_specs=pl.BlockSpec((2000, 256), lambda i: (i, 0)),
        out_shape=jax.ShapeDtypeStruct((N, 256), jnp.float32),
    )(h, Wpq)


_EB = 2000


def _mlp_body(u_ref, t_ref, w_ref, mb1_ref, w2_ref, mb2_ref, o_ref):
    u = u_ref[0]
    t = t_ref[0]
    hid = jnp.maximum(u + jnp.dot(t, w_ref[...],
                                  preferred_element_type=jnp.float32)
                      + mb1_ref[...], 0.0)
    lg = jnp.dot(hid, w2_ref[...], preferred_element_type=jnp.float32)
    o_ref[0, 0, :] = lg[:, 0] + mb2_ref[0]


def _mlp_part(u, t3, w_t, mb1, mW2, mb2, kpt, koff):
    nb = NW * kpt
    u3 = u.reshape(nb, _EB, 128)
    out = pl.pallas_call(
        _mlp_body,
        grid=(nb,),
        in_specs=[
            pl.BlockSpec((1, _EB, 128), lambda i: (i, 0, 0)),
            pl.BlockSpec((1, _EB, 128),
                         lambda i: (i // kpt * 5 + koff + i % kpt, 0, 0)),
            pl.BlockSpec((128, 128), lambda i: (0, 0)),
            pl.BlockSpec((128,), lambda i: (0,)),
            pl.BlockSpec((128, 1), lambda i: (0, 0)),
            pl.BlockSpec((1,), lambda i: (0,)),
        ],
        out_specs=pl.BlockSpec((1, 1, _EB), lambda i: (i, 0, 0)),
        out_shape=jax.ShapeDtypeStruct((nb, 1, _EB), jnp.float32),
    )(u3, t3, w_t, mb1, mW2, mb2)
    return out.reshape(NW, kpt * _EB)


def kernel(edge_index, edge_type, e_text_emb, node_emb, Wr1, Wroot1, b1,
           Wr2, Wroot2, b2, mW1, mb1, mW2, mb2):
    src = edge_index[0]
    dst = edge_index[1]
    et = edge_type

    key3 = (dst * R + et).astype(jnp.int32).reshape(NW, NB, BE)
    iy = (et * N + src).astype(jnp.int32)
    pk3 = (iy | (dst.astype(jnp.int32) << 17)).reshape(NW, NB, BE)
    ip3 = (2 * src).astype(jnp.int32).reshape(NW, NB, BE)
    iq3 = (2 * dst + 1).astype(jnp.int32).reshape(NW, NB, BE)

    hpart = _hist_k(key3)
    invc = _inv_k(hpart)
    scale3 = _scale_k(key3, invc)


    x0 = node_emb
    Y1 = _ymul_k(x0, Wr1)
    pk4 = pk3.reshape(NW, 5, 25, BE)
    sc4 = scale3.reshape(NW, 5, 25, BE)
    a1 = _agg_k(Y1, pk4, sc4)
    h1 = _dense_k(a1, x0, Wroot1, b1)

    Y2 = _ymul_k(h1, Wr2)
    a2 = _agg_k(Y2, pk4, sc4)
    h = _dense_k(a2, h1, Wroot2, b2)

    Wpq = jnp.concatenate([mW1[:128], mW1[128:256]], axis=1)
    PQ2 = _pq_k(h, Wpq).reshape(2 * N, 128)
    ip4 = ip3.reshape(NW, 5, 25, BE)
    iq4 = iq3.reshape(NW, 5, 25, BE)
    t3 = e_text_emb.reshape(NW * 5, _EB, 128)
    w_t = mW1[256:]
    lparts = []
    for k in range(5):
        uk = _gath_p[k](PQ2, ip4, iq4)
        lparts.append(_mlp_part(uk, t3, w_t, mb1, mW2, mb2, 1, k))
    logits = jnp.concatenate(lparts, axis=1).reshape(E)
    return (logits, h)


# speedup vs baseline: 6.7350x; 1.0164x over previous
import functools
import jax
import jax.numpy as jnp
from jax import lax
from jax.experimental import pallas as pl
from jax.experimental.pallas import tpu as pltpu
from jax.experimental.pallas import tpu_sc as plsc

N = 10000
R = 8
E = 320000
NW = 32
EPW = E // NW
NB = 125
BE = 80
HB = 81920
NP = 10240

_mesh = plsc.VectorSubcoreMesh(core_axis_name="c", subcore_axis_name="s")


def _wid():
    return lax.axis_index("c") * 16 + lax.axis_index("s")


@functools.partial(
    pl.kernel,
    out_type=jax.ShapeDtypeStruct((2, HB), jnp.float32),
    mesh=_mesh,
    scratch_types=[
        pltpu.VMEM((NB, BE), jnp.int32),
        pltpu.VMEM((BE,), jnp.float32),
        pltpu.VMEM((5120,), jnp.float32),
        pltpu.VMEM_SHARED((HB,), jnp.float32),
    ],
)
def _hist_k(key_hbm, out_hbm, kb, ones, zb, hist):
    c = lax.axis_index("c")
    s = lax.axis_index("s")
    w = _wid()

    def fill_ones(i, _):
        ones[pl.ds(i * 16, 16)] = jnp.ones((16,), jnp.float32)
        return 0

    lax.fori_loop(0, BE // 16, fill_ones, 0)

    def fill_z(i, _):
        zb[pl.ds(i * 16, 16)] = jnp.zeros((16,), jnp.float32)
        return 0

    lax.fori_loop(0, 5120 // 16, fill_z, 0)
    pltpu.sync_copy(zb, hist.at[pl.ds(s * 5120, 5120)])
    plsc.subcore_barrier()

    pltpu.sync_copy(key_hbm.at[w], kb)

    def body(j, _):
        pltpu.sync_copy(ones, hist.at[kb.at[j]], add=True)
        return 0

    lax.fori_loop(0, NB, body, 0)
    plsc.subcore_barrier()
    pltpu.sync_copy(hist.at[pl.ds(s * 5120, 5120)],
                    out_hbm.at[c, pl.ds(s * 5120, 5120)])


@functools.partial(
    pl.kernel,
    out_type=jax.ShapeDtypeStruct((NW, NB, BE), jnp.float32),
    mesh=_mesh,
    scratch_types=[
        pltpu.VMEM((NB, BE), jnp.int32),
        pltpu.VMEM((NB, BE), jnp.float32),
        pltpu.SemaphoreType.DMA,
    ],
)
def _scale_k(key_hbm, invc_hbm, out_hbm, kb, sb, sem):
    w = _wid()
    pltpu.sync_copy(key_hbm.at[w], kb)

    def chunk(cc, _):
        j0 = cc * 25
        descs = []
        for b in range(25):
            descs.append(pltpu.async_copy(
                invc_hbm.at[kb.at[j0 + b]], sb.at[j0 + b], sem))
        for b in range(25):
            descs[b].wait()
        return 0

    lax.fori_loop(0, 5, chunk, 0)
    pltpu.sync_copy(sb, out_hbm.at[w])


@functools.partial(
    pl.kernel,
    out_type=jax.ShapeDtypeStruct((2, NP, 128), jnp.float32),
    mesh=_mesh,
    scratch_types=[
        pltpu.VMEM((25, BE), jnp.int32),
        pltpu.VMEM((25, BE), jnp.float32),
        pltpu.VMEM((3, BE), jnp.int32),
        pltpu.VMEM((3, BE), jnp.int32),
        pltpu.VMEM((3, BE, 128), jnp.float32),
        pltpu.VMEM((40, 128), jnp.float32),
        pltpu.VMEM_SHARED((NP, 128), jnp.float32),
        pltpu.SemaphoreType.DMA,
        pltpu.SemaphoreType.DMA,
    ],
)
def _agg_k(y_hbm, pk_hbm, sc_hbm, out_hbm,
           pkb, sb, iyblk, dstblk, rb, zb, acc, sem, ssc):
    c = lax.axis_index("c")
    s = lax.axis_index("s")
    w = _wid()

    def fill_z(i, _):
        row = i // 8
        f = i % 8
        zb[row, pl.ds(f * 16, 16)] = jnp.zeros((16,), jnp.float32)
        return 0

    lax.fori_loop(0, 40 * 8, fill_z, 0)
    zcs = []
    for i in range(16):
        zcs.append(pltpu.async_copy(
            zb, acc.at[pl.ds(s * 640 + i * 40, 40)], sem))
    for i in range(16):
        zcs[i].wait()
    plsc.subcore_barrier()

    def unpack(j, buf):
        for i in range(BE // 16):
            v = pkb[j, pl.ds(i * 16, 16)]
            iyblk[buf, pl.ds(i * 16, 16)] = v & 0x1FFFF
            dstblk[buf, pl.ds(i * 16, 16)] = lax.shift_right_logical(v, 17)

    def issue(buf):
        return pltpu.async_copy(y_hbm.at[iyblk.at[buf]], rb.at[buf], sem)

    def wait_g():
        pltpu.make_async_copy(y_hbm.at[iyblk.at[0]], rb.at[0], sem).wait()

    def scale_scatter(j, buf):
        def escale(i, _):
            sc16 = sb[j, pl.ds(i * 16, 16)]
            for ee in range(16):
                e = i * 16 + ee
                sc = sc16[ee]
                for f in range(8):
                    rb[buf, e, pl.ds(f * 16, 16)] = (
                        rb[buf, e, pl.ds(f * 16, 16)] * sc)
            return 0

        lax.fori_loop(0, BE // 16, escale, 0)
        pltpu.async_copy(rb.at[buf], acc.at[dstblk.at[buf]], ssc, add=True)

    def wait_sc():
        pltpu.make_async_copy(rb.at[0], acc.at[dstblk.at[0]], ssc).wait()

    def chunk(cc, _):
        pltpu.sync_copy(pk_hbm.at[w, cc], pkb)
        pltpu.sync_copy(sc_hbm.at[w, cc], sb)
        unpack(0, 0)
        issue(0)
        for b in range(1, 25):
            if b >= 3:
                wait_sc()
            unpack(b, b % 3)
            issue(b % 3)
            wait_g()
            scale_scatter(b - 1, (b - 1) % 3)
        wait_g()
        scale_scatter(24, 0)
        for _i in range(3):
            wait_sc()
        return 0

    lax.fori_loop(0, 5, chunk, 0)
    plsc.subcore_barrier()
    pltpu.sync_copy(acc.at[pl.ds(s * 640, 640)],
                    out_hbm.at[c, pl.ds(s * 640, 640)])


def _make_gath(nchunk, off, ne):
    @functools.partial(
        pl.kernel,
        out_type=jax.ShapeDtypeStruct((NW * ne, 128), jnp.float32),
        mesh=_mesh,
        scratch_types=[
            pltpu.VMEM((25, BE), jnp.int32),
            pltpu.VMEM((25, BE), jnp.int32),
            pltpu.VMEM((5, BE, 128), jnp.float32),
            pltpu.VMEM((5, BE, 128), jnp.float32),
            pltpu.SemaphoreType.DMA,
            pltpu.SemaphoreType.DMA,
            pltpu.SemaphoreType.DMA,
        ],
    )
    def _gath_k(pq_hbm, ip_hbm, iq_hbm, u_hbm,
                ipb, iqb, pb, qb, sp, sq, sw):
        w = _wid()

        def chunk(cc, _):
            pltpu.sync_copy(ip_hbm.at[w, off + cc], ipb)
            pltpu.sync_copy(iq_hbm.at[w, off + cc], iqb)

            def sub(dd, _):
                j0 = dd * 5
                cps = []
                cqs = []
                for b in range(5):
                    cps.append(pltpu.async_copy(
                        pq_hbm.at[ipb.at[j0 + b]], pb.at[b], sp))
                    cqs.append(pltpu.async_copy(
                        pq_hbm.at[iqb.at[j0 + b]], qb.at[b], sq))
                cws = []
                for b in range(5):
                    cps[b].wait()
                    cqs[b].wait()

                    def eadd(e, _):
                        for f in range(8):
                            pb[b, e, pl.ds(f * 16, 16)] = (
                                pb[b, e, pl.ds(f * 16, 16)]
                                + qb[b, e, pl.ds(f * 16, 16)])
                        return 0

                    lax.fori_loop(0, BE, eadd, 0)
                    base = w * ne + (cc * 25 + j0 + b) * BE
                    cws.append(pltpu.async_copy(
                        pb.at[b], u_hbm.at[pl.ds(base, BE)], sw))
                for b in range(5):
                    cws[b].wait()
                return 0

            lax.fori_loop(0, 5, sub, 0)
            return 0

        lax.fori_loop(0, nchunk, chunk, 0)

    return _gath_k


_gath_p = [_make_gath(1, k, 2000) for k in range(5)]


def _inv_body(p_ref, o_ref):
    o_ref[...] = 1.0 / jnp.maximum(p_ref[0] + p_ref[1], 1.0)


def _inv_k(hpart):
    h3 = hpart.reshape(2, 640, 128)
    out = pl.pallas_call(
        _inv_body,
        grid=(5,),
        in_specs=[pl.BlockSpec((2, 128, 128), lambda i: (0, i, 0))],
        out_specs=pl.BlockSpec((128, 128), lambda i: (i, 0)),
        out_shape=jax.ShapeDtypeStruct((640, 128), jnp.float32),
    )(h3)
    return out.reshape(HB)


def _ymul_body(x_ref, w_ref, o_ref):
    o_ref[0] = jnp.dot(x_ref[...], w_ref[0],
                       preferred_element_type=jnp.float32)


def _ymul_k(x, Wr):
    out = pl.pallas_call(
        _ymul_body,
        grid=(R, 5),
        in_specs=[
            pl.BlockSpec((2000, 128), lambda r, i: (i, 0)),
            pl.BlockSpec((1, 128, 128), lambda r, i: (r, 0, 0)),
        ],
        out_specs=pl.BlockSpec((1, 2000, 128), lambda r, i: (r, i, 0)),
        out_shape=jax.ShapeDtypeStruct((R, N, 128), jnp.float32),
    )(x, Wr)
    return out.reshape(R * N, 128)


def _dense_body(a_ref, x_ref, wr_ref, b_ref, o_ref):
    o_ref[...] = jnp.maximum(
        a_ref[0] + a_ref[1]
        + jnp.dot(x_ref[...], wr_ref[...], preferred_element_type=jnp.float32)
        + b_ref[...], 0.0)


def _dense_k(apart, x, Wroot, b):
    return pl.pallas_call(
        _dense_body,
        grid=(5,),
        in_specs=[
            pl.BlockSpec((2, 2000, 128), lambda i: (0, i, 0)),
            pl.BlockSpec((2000, 128), lambda i: (i, 0)),
            pl.BlockSpec((128, 128), lambda i: (0, 0)),
            pl.BlockSpec((128,), lambda i: (0,)),
        ],
        out_specs=pl.BlockSpec((2000, 128), lambda i: (i, 0)),
        out_shape=jax.ShapeDtypeStruct((N, 128), jnp.float32),
    )(apart, x, Wroot, b)


def _pq_body(x_ref, w_ref, o_ref):
    o_ref[...] = jnp.dot(x_ref[...], w_ref[...],
                         preferred_element_type=jnp.float32)


def _pq_k(h, Wpq):
    return pl.pallas_call(
        _pq_body,
        grid=(5,),
        in_specs=[
            pl.BlockSpec((2000, 128), lambda i: (i, 0)),
            pl.BlockSpec((128, 256), lambda i: (0, 0)),
        ],
        out_specs=pl.BlockSpec((2000, 256), lambda i: (i, 0)),
        out_shape=jax.ShapeDtypeStruct((N, 256), jnp.float32),
    )(h, Wpq)


_EB = 2000


def _mlp_body(u_ref, t_ref, w_ref, mb1_ref, w2_ref, mb2_ref, o_ref):
    u = u_ref[0]
    t = t_ref[0]
    hid = jnp.maximum(u + jnp.dot(t, w_ref[...],
                                  preferred_element_type=jnp.float32)
                      + mb1_ref[...], 0.0)
    lg = jnp.dot(hid, w2_ref[...], preferred_element_type=jnp.float32)
    o_ref[0, 0, :] = lg[:, 0] + mb2_ref[0]


def _mlp_part(u, t3, w_t, mb1, mW2, mb2, kpt, koff):
    nb = NW * kpt
    u3 = u.reshape(nb, _EB, 128)
    out = pl.pallas_call(
        _mlp_body,
        grid=(nb,),
        in_specs=[
            pl.BlockSpec((1, _EB, 128), lambda i: (i, 0, 0)),
            pl.BlockSpec((1, _EB, 128),
                         lambda i: (i // kpt * 5 + koff + i % kpt, 0, 0)),
            pl.BlockSpec((128, 128), lambda i: (0, 0)),
            pl.BlockSpec((128,), lambda i: (0,)),
            pl.BlockSpec((128, 1), lambda i: (0, 0)),
            pl.BlockSpec((1,), lambda i: (0,)),
        ],
        out_specs=pl.BlockSpec((1, 1, _EB), lambda i: (i, 0, 0)),
        out_shape=jax.ShapeDtypeStruct((nb, 1, _EB), jnp.float32),
    )(u3, t3, w_t, mb1, mW2, mb2)
    return out.reshape(NW, kpt * _EB)


def kernel(edge_index, edge_type, e_text_emb, node_emb, Wr1, Wroot1, b1,
           Wr2, Wroot2, b2, mW1, mb1, mW2, mb2):
    src = edge_index[0]
    dst = edge_index[1]
    et = edge_type

    key3 = (dst * R + et).astype(jnp.int32).reshape(NW, NB, BE)
    iy = (et * N + src).astype(jnp.int32)
    pk3 = (iy | (dst.astype(jnp.int32) << 17)).reshape(NW, NB, BE)
    ip3 = (2 * src).astype(jnp.int32).reshape(NW, NB, BE)
    iq3 = (2 * dst + 1).astype(jnp.int32).reshape(NW, NB, BE)

    hpart = _hist_k(key3)
    invc = _inv_k(hpart)
    scale3 = _scale_k(key3, invc)


    x0 = node_emb
    Y1 = _ymul_k(x0, Wr1)
    pk4 = pk3.reshape(NW, 5, 25, BE)
    sc4 = scale3.reshape(NW, 5, 25, BE)
    a1 = _agg_k(Y1, pk4, sc4)
    h1 = _dense_k(a1, x0, Wroot1, b1)

    Y2 = _ymul_k(h1, Wr2)
    a2 = _agg_k(Y2, pk4, sc4)
    h = _dense_k(a2, h1, Wroot2, b2)

    Wpq = jnp.concatenate([mW1[:128], mW1[128:256]], axis=1)
    PQ2 = _pq_k(h, Wpq).reshape(2 * N, 128)
    ip4 = ip3.reshape(NW, 5, 25, BE)
    iq4 = iq3.reshape(NW, 5, 25, BE)
    t3 = e_text_emb.reshape(NW * 5, _EB, 128)
    w_t = mW1[256:]
    lparts = []
    for k in range(5):
        uk = _gath_p[k](PQ2, ip4, iq4)
        lparts.append(_mlp_part(uk, t3, w_t, mb1, mW2, mb2, 1, k))
    logits = jnp.concatenate(lparts, axis=1).reshape(E)
    return (logits, h)
